# baseline (speedup 1.0000x reference)
import functools

import jax
import jax.numpy as jnp
from jax import lax
from jax.experimental import pallas as pl
from jax.experimental.pallas import tpu as pltpu
from jax.experimental.pallas import tpu_sc as plsc

N_VOCAB = 100000
D = 64
BATCH = 1024
SEQ = 200
OUT_SEQ = SEQ + 1
NC = 2
NS = 16
NW = NC * NS
SP = (OUT_SEQ + 1) // 2
BC = 128
UPS = BATCH // BC
GRP = 5

SPA = 50
SPB = SP - SPA
UNITS_A = 100 * UPS
UNITS_B = 960
UPW_A = UNITS_A // NW
UPW_B = UNITS_B // NW
ROWS_A = SPA * BATCH
ROWS_B = SPB * BATCH

_mesh = plsc.VectorSubcoreMesh(core_axis_name="c", subcore_axis_name="s")


def _make_gather(s_base, upw, mid_rows):
    gpw = upw // GRP

    def body(ids_hbm, table_hbm, out_hbm, idx_v, buf0, buf1, sem0, sem1):
        wid = lax.axis_index("s") * NC + lax.axis_index("c")
        u_base = wid * upw

        pltpu.sync_copy(ids_hbm.at[pl.ds(u_base, upw)], idx_v)

        def issue_group(g, buf, sem):
            for j in range(GRP):
                dst = buf.at[pl.ds(j * BC, BC)]
                pltpu.async_copy(table_hbm.at[idx_v.at[g * GRP + j]], dst, sem)

        def drain_group(buf, sem):
            pltpu.make_async_copy(table_hbm.at[pl.ds(0, GRP * BC)], buf, sem).wait()

        def write_group(g, buf):
            for j in range(GRP):
                u = u_base + g * GRP + j
                s = jnp.minimum(s_base + u // UPS, SEQ + 1)
                row0 = (s // 2 - s_base // 2) * BATCH + (u % UPS) * BC
                dst = out_hbm.at[pl.ds(row0, BC), pl.ds((s % 2) * D, D)]
                pltpu.sync_copy(buf.at[pl.ds(j * BC, BC)], dst)

        issue_group(0, buf0, sem0)

        def loop_body(i, carry):
            g = 2 * i
            issue_group(g + 1, buf1, sem1)
            drain_group(buf0, sem0)
            write_group(g, buf0)

            @pl.when(g + 2 < gpw)
            def _():
                issue_group(g + 2, buf0, sem0)

            drain_group(buf1, sem1)
            write_group(g + 1, buf1)
            return carry

        lax.fori_loop(0, gpw // 2, loop_body, 0)
        if gpw % 2:
            drain_group(buf0, sem0)
            write_group(gpw - 1, buf0)

    return pl.kernel(
        body,
        mesh=_mesh,
        out_type=jax.ShapeDtypeStruct((mid_rows, 2 * D), jnp.float32),
        scratch_types=[
            pltpu.VMEM((upw, BC), jnp.int32),
            pltpu.VMEM((GRP * BC, D), jnp.float32),
            pltpu.VMEM((GRP * BC, D), jnp.float32),
            pltpu.SemaphoreType.DMA,
            pltpu.SemaphoreType.DMA,
        ],
        compiler_params=pltpu.CompilerParams(use_tc_tiling_on_sc=False),
    )


_gather_a = _make_gather(0, UPW_A, ROWS_A)
_gather_b = _make_gather(100, UPW_B, ROWS_B)


def _table_body(xa_ref, xb_ref, y_ref):
    y_ref[...] = jnp.concatenate([xa_ref[...], xb_ref[...]], axis=0).transpose()


PAIR = 49 * 1024

_tc_table = pl.pallas_call(
    _table_body,
    grid=(49,),
    in_specs=[
        pl.BlockSpec((D, 1024), lambda i: (0, i)),
        pl.BlockSpec((D, 1024), lambda i: (0, i + 49)),
    ],
    out_specs=pl.BlockSpec((1024, 2 * D), lambda i: (i, 0)),
    out_shape=jax.ShapeDtypeStruct((PAIR, 2 * D), jnp.float32),
)


def _transpose_body(x_ref, y_ref):
    xt = x_ref[...].transpose()
    y_ref[0] = xt[:D]
    y_ref[1] = xt[D:]


def _transpose_body_aliased(x_ref, yin_ref, y_ref):
    del yin_ref
    _transpose_body(x_ref, y_ref)


_tc_transpose_a = pl.pallas_call(
    _transpose_body,
    grid=(SPA,),
    in_specs=[pl.BlockSpec((BATCH, 2 * D), lambda i: (i, 0))],
    out_specs=pl.BlockSpec((2, D, BATCH), lambda i: (i, 0, 0)),
    out_shape=jax.ShapeDtypeStruct((OUT_SEQ, D, BATCH), jnp.float32),
)

_tc_transpose_b = pl.pallas_call(
    _transpose_body_aliased,
    grid=(SPB,),
    in_specs=[
        pl.BlockSpec((BATCH, 2 * D), lambda i: (i, 0)),
        pl.BlockSpec(memory_space=pl.ANY),
    ],
    out_specs=pl.BlockSpec((2, D, BATCH), lambda i: (i + SPA, 0, 0)),
    out_shape=jax.ShapeDtypeStruct((OUT_SEQ, D, BATCH), jnp.float32),
    input_output_aliases={1: 0},
)


def kernel(input_ids, embedding_weight):
    ids_v = jnp.where(input_ids < PAIR, 2 * input_ids, 2 * input_ids - (2 * PAIR - 1))
    ids2 = ids_v.T.reshape(SEQ * UPS, BC)
    zid = jnp.zeros((8, BC), jnp.int32)
    ids_a = jnp.concatenate([zid, ids2[: 99 * UPS]], axis=0)
    ids_b = jnp.concatenate(
        [ids2[99 * UPS :], jnp.zeros((UNITS_B - 101 * UPS, BC), jnp.int32)], axis=0
    )
    xt = embedding_weight.T
    wt_lin = _tc_table(xt, xt).reshape(2 * PAIR, D)
    mid_a = _gather_a(ids_a, wt_lin)
    mid_b = _gather_b(ids_b, wt_lin)
    y = _tc_transpose_a(mid_a)
    y = _tc_transpose_b(mid_b, y)
    return jnp.transpose(y, (2, 0, 1))


# speedup vs baseline: 3.3682x; 3.3682x over previous
import functools

import jax
import jax.numpy as jnp
from jax import lax
from jax.experimental import pallas as pl
from jax.experimental.pallas import tpu as pltpu
from jax.experimental.pallas import tpu_sc as plsc

N_VOCAB = 100000
D = 64
BATCH = 1024
SEQ = 200
OUT_SEQ = SEQ + 1
NC = 2
NS = 16
NW = NC * NS
SP = (OUT_SEQ + 1) // 2
BC = 128
UPS = BATCH // BC
GRP = 5

SPA = 50
SPB = SP - SPA
REAL_A = 99 * UPS
REAL_B = 101 * UPS
UNITS_A = 800
UNITS_B = 960
UPW_A = UNITS_A // NW
UPW_B = UNITS_B // NW
ROWS_A = SPA * BATCH + (UNITS_A - REAL_A) * BC
ROWS_B = SPB * BATCH + (UNITS_B - REAL_B) * BC

_mesh = plsc.VectorSubcoreMesh(core_axis_name="c", subcore_axis_name="s")


def _make_gather(s_base, upw, mid_rows, real_units):
    gpw = upw // GRP
    scrap_base = mid_rows - (upw * NW - real_units) * BC

    def body(ids_hbm, table_hbm, out_hbm, idx_v, buf0, buf1, sem0, sem1):
        wid = lax.axis_index("s") * NC + lax.axis_index("c")
        u_base = wid * upw

        pltpu.sync_copy(ids_hbm.at[pl.ds(u_base, upw)], idx_v)

        def issue_group(g, buf, sem):
            for j in range(GRP):
                dst = buf.at[pl.ds(j * BC, BC)]
                pltpu.async_copy(table_hbm.at[idx_v.at[g * GRP + j]], dst, sem)

        def drain_group(buf, sem):
            pltpu.make_async_copy(table_hbm.at[pl.ds(0, GRP * BC)], buf, sem).wait()

        def write_group(g, buf):
            for j in range(GRP):
                u = u_base + g * GRP + j
                s = jnp.minimum(s_base + u // UPS, SEQ)
                real_row = (s // 2 - s_base // 2) * BATCH + (u % UPS) * BC
                row0 = jnp.where(
                    u < real_units, real_row, scrap_base + (u - real_units) * BC
                )
                dst = out_hbm.at[pl.ds(row0, BC), pl.ds((s % 2) * D, D)]
                pltpu.sync_copy(buf.at[pl.ds(j * BC, BC)], dst)

        issue_group(0, buf0, sem0)

        def loop_body(i, carry):
            g = 2 * i
            issue_group(g + 1, buf1, sem1)
            drain_group(buf0, sem0)
            write_group(g, buf0)

            @pl.when(g + 2 < gpw)
            def _():
                issue_group(g + 2, buf0, sem0)

            drain_group(buf1, sem1)
            write_group(g + 1, buf1)
            return carry

        lax.fori_loop(0, gpw // 2, loop_body, 0)
        if gpw % 2:
            drain_group(buf0, sem0)
            write_group(gpw - 1, buf0)

    return pl.kernel(
        body,
        mesh=_mesh,
        out_type=jax.ShapeDtypeStruct((mid_rows, 2 * D), jnp.float32),
        scratch_types=[
            pltpu.VMEM((upw, BC), jnp.int32),
            pltpu.VMEM((GRP * BC, D), jnp.float32),
            pltpu.VMEM((GRP * BC, D), jnp.float32),
            pltpu.SemaphoreType.DMA,
            pltpu.SemaphoreType.DMA,
        ],
        compiler_params=pltpu.CompilerParams(use_tc_tiling_on_sc=False),
    )


_gather_a = _make_gather(1, UPW_A, ROWS_A, REAL_A)
_gather_b = _make_gather(100, UPW_B, ROWS_B, REAL_B)


def _table_body(xa_ref, xb_ref, y_ref):
    y_ref[...] = jnp.concatenate([xa_ref[...], xb_ref[...]], axis=0).transpose()


PAIR = 49 * 1024

_tc_table = pl.pallas_call(
    _table_body,
    grid=(49,),
    in_specs=[
        pl.BlockSpec((D, 1024), lambda i: (0, i)),
        pl.BlockSpec((D, 1024), lambda i: (0, i + 49)),
    ],
    out_specs=pl.BlockSpec((1024, 2 * D), lambda i: (i, 0)),
    out_shape=jax.ShapeDtypeStruct((PAIR, 2 * D), jnp.float32),
)


def _transpose_body(x_ref, y_ref):
    xt = x_ref[...].transpose()
    y_ref[0] = xt[:D]
    y_ref[1] = xt[D:]


def _transpose_body_aliased(x_ref, yin_ref, y_ref):
    del yin_ref
    _transpose_body(x_ref, y_ref)


def _transpose_body_a(x_ref, bos_ref, y_ref):
    xt = x_ref[...].transpose()

    @pl.when(pl.program_id(0) == 0)
    def _():
        y_ref[0] = jnp.broadcast_to(bos_ref[...].transpose(), (D, BATCH))

    @pl.when(pl.program_id(0) > 0)
    def _():
        y_ref[0] = xt[:D]

    y_ref[1] = xt[D:]


_tc_transpose_a = pl.pallas_call(
    _transpose_body_a,
    grid=(SPA,),
    in_specs=[
        pl.BlockSpec((BATCH, 2 * D), lambda i: (i, 0)),
        pl.BlockSpec((1, D), lambda i: (0, 0)),
    ],
    out_specs=pl.BlockSpec((2, D, BATCH), lambda i: (i, 0, 0)),
    out_shape=jax.ShapeDtypeStruct((OUT_SEQ, D, BATCH), jnp.float32),
)

_tc_transpose_b = pl.pallas_call(
    _transpose_body_aliased,
    grid=(SPB,),
    in_specs=[
        pl.BlockSpec((BATCH, 2 * D), lambda i: (i, 0)),
        pl.BlockSpec(memory_space=pl.ANY),
    ],
    out_specs=pl.BlockSpec((2, D, BATCH), lambda i: (i + SPA, 0, 0)),
    out_shape=jax.ShapeDtypeStruct((OUT_SEQ, D, BATCH), jnp.float32),
    input_output_aliases={1: 0},
)


def kernel(input_ids, embedding_weight):
    ids_v = jnp.where(input_ids < PAIR, 2 * input_ids, 2 * input_ids - (2 * PAIR - 1))
    ids2 = ids_v.T.reshape(SEQ * UPS, BC)
    dum = jnp.arange(BC, dtype=jnp.int32) * 2
    ids_a = jnp.concatenate(
        [ids2[:REAL_A], jnp.broadcast_to(dum, (UNITS_A - REAL_A, BC))], axis=0
    )
    ids_b = jnp.concatenate(
        [ids2[REAL_A:], jnp.broadcast_to(dum, (UNITS_B - REAL_B, BC))], axis=0
    )
    xt = embedding_weight.T
    wt_lin = _tc_table(xt, xt).reshape(2 * PAIR, D)
    mid_a = _gather_a(ids_a, wt_lin)
    mid_b = _gather_b(ids_b, wt_lin)
    y = _tc_transpose_a(mid_a, wt_lin[:1])
    y = _tc_transpose_b(mid_b, y)
    return jnp.transpose(y, (2, 0, 1))
